# baseline (speedup 1.0000x reference)
import functools

import jax
import jax.numpy as jnp
import numpy as np
from jax import lax
from jax.experimental import pallas as pl
from jax.experimental.pallas import tpu as pltpu
from jax.experimental.pallas import tpu_sc as plsc

N = 10000
K = 30
NP = 10240
NE = NP * K
NCOEF = 9
HC = 32
CTOT = 35
NHEADS = 8
VCH = 8
D = 384

KNN_R = 256
KNN_CCH = 1024
KNN_NCH = NP // KNN_CCH

AT_R = 64
AT_E = AT_R * K

SC_NC = 2
SC_NS = 16
SC_NW = SC_NC * SC_NS
SC_PER_W = NE // SC_NW
SC_CH = 320
SC_NIT = SC_PER_W // SC_CH

_SIGMA = np.float32(20.0 / 16)
_IDEAL = np.array([[-0.525, 1.363, 0.0], [0.0, 0.0, 0.0], [1.526, 0.0, 0.0]],
                  dtype=np.float32)


def _knn_body(tb_ref, tT_ref, bb_ref, bT_ref, out_ref, d2_ref, idxs_ref,
              cmin_ref, sub_ref, idx_scr):
    pid = pl.program_id(0)
    NSUB = KNN_CCH // 128
    xb = tb_ref[...]
    bb = bb_ref[...]
    rowid = lax.broadcasted_iota(jnp.int32, (KNN_R, 1), 0) + pid * KNN_R
    liota = lax.broadcasted_iota(jnp.int32, (KNN_R, KNN_CCH), 1)
    piota = lax.broadcasted_iota(jnp.int32, (KNN_R, 128), 1)
    siota = lax.broadcasted_iota(jnp.int32, (KNN_R, NSUB), 1)

    def fill(c, _):
        x0 = xb[:, 0:1] - tT_ref[c, 0:1, :]
        x1 = xb[:, 1:2] - tT_ref[c, 1:2, :]
        x2 = xb[:, 2:3] - tT_ref[c, 2:3, :]
        d2 = (x0 * x0 + x1 * x1) + x2 * x2
        bm = bb != bT_ref[c, 0:1, :]
        d2 = jnp.where(bm, 1e9, d2)
        d2 = jnp.where(liota == rowid - c * KNN_CCH, 1e9, d2)
        sms = []
        for j in range(NSUB):
            piece = d2[:, j * 128:(j + 1) * 128]
            d2_ref[c * NSUB + j] = piece
            sms.append(jnp.min(piece, axis=1, keepdims=True))
        sm = jnp.concatenate(sms, axis=1)
        sub_ref[c] = sm
        cmin_ref[c] = jnp.min(sm, axis=1, keepdims=True)
        return 0

    lax.fori_loop(0, KNN_NCH, fill, 0)

    ciota = lax.broadcasted_iota(jnp.int32, (KNN_NCH, KNN_R, 1), 0)

    def extract_one(i, _):
        cmin3 = cmin_ref[...]
        m = jnp.min(cmin3, axis=0)
        cbest = jnp.min(jnp.where(cmin3 == m[None], ciota, 1 << 30), axis=0)

        def scan(c, _):
            @pl.when(jnp.any(cbest == c))
            def _():
                rowsel = cbest == c
                sm = sub_ref[c]
                sbest = jnp.min(
                    jnp.where((sm == m) & rowsel, siota, 1 << 30),
                    axis=1, keepdims=True)

                def scan_sub(j, _):
                    @pl.when(jnp.any(sbest == j))
                    def _():
                        psel = sbest == j
                        piece = d2_ref[c * NSUB + j]
                        lidx = jnp.min(
                            jnp.where((piece == m) & psel, piota, 1 << 30),
                            axis=1, keepdims=True)
                        pn = jnp.where(piota == lidx, 2e9, piece)
                        d2_ref[c * NSUB + j] = pn
                        npm = jnp.min(pn, axis=1, keepdims=True)
                        smv = jnp.where((siota == j) & psel, npm, sub_ref[c])
                        sub_ref[c] = smv
                        cmin_ref[c] = jnp.min(smv, axis=1, keepdims=True)
                        idx_scr[...] = jnp.where(
                            psel, lidx + (c * KNN_CCH + j * 128),
                            idx_scr[...])
                    return 0

                lax.fori_loop(0, NSUB, scan_sub, 0)
            return 0

        lax.fori_loop(0, KNN_NCH, scan, 0)
        idxs_ref[i] = idx_scr[...]
        return 0

    lax.fori_loop(0, K, extract_one, 0)
    for k in range(K):
        out_ref[:, k:k + 1] = idxs_ref[k]


def _knn(trans_pad, transT3, batch2d, batchT3):
    return pl.pallas_call(
        _knn_body,
        grid=(NP // KNN_R,),
        in_specs=[
            pl.BlockSpec((KNN_R, 3), lambda i: (i, 0)),
            pl.BlockSpec((KNN_NCH, 3, KNN_CCH), lambda i: (0, 0, 0)),
            pl.BlockSpec((KNN_R, 1), lambda i: (i, 0)),
            pl.BlockSpec((KNN_NCH, 1, KNN_CCH), lambda i: (0, 0, 0)),
        ],
        out_specs=pl.BlockSpec((KNN_R, K), lambda i: (i, 0)),
        out_shape=jax.ShapeDtypeStruct((NP, K), jnp.int32),
        scratch_shapes=[pltpu.VMEM((NP // 128, KNN_R, 128), jnp.float32),
                        pltpu.VMEM((K, KNN_R, 1), jnp.int32),
                        pltpu.VMEM((KNN_NCH, KNN_R, 1), jnp.float32),
                        pltpu.VMEM((KNN_NCH, KNN_R, KNN_CCH // 128),
                                   jnp.float32),
                        pltpu.VMEM((KNN_R, 1), jnp.int32)],
    )(trans_pad, transT3, batch2d, batchT3)


def _sc_gather_body(table_hbm, idx_hbm, out_hbm, idx_v, rows_v, sem):
    wid = lax.axis_index("s") * SC_NC + lax.axis_index("c")
    base = wid * SC_PER_W

    def body(it, _):
        off = base + it * SC_CH
        pltpu.sync_copy(idx_hbm.at[pl.ds(off, SC_CH)], idx_v)
        pltpu.async_copy(table_hbm.at[idx_v], rows_v, sem).wait()
        pltpu.sync_copy(rows_v, out_hbm.at[pl.ds(off, SC_CH)])
        return 0

    lax.fori_loop(0, SC_NIT, body, 0)


def _sc_gather(table, idx_flat):
    mesh = plsc.VectorSubcoreMesh(core_axis_name="c", subcore_axis_name="s")
    f = functools.partial(
        pl.kernel,
        mesh=mesh,
        out_type=jax.ShapeDtypeStruct((NE, D), jnp.float32),
        scratch_types=[
            pltpu.VMEM((SC_CH,), jnp.int32),
            pltpu.VMEM((SC_CH, D), jnp.float32),
            pltpu.SemaphoreType.DMA,
        ],
    )(_sc_gather_body)
    return f(table, idx_flat)


def _attn_body(g_ref, tab_ref, ne_ref, sf_ref,
               wadst_ref, wasrc_ref, wedge_ref, bedge_ref, waeh_ref,
               wval_ref, phead_ref, wproj_ref, wff1_ref, tgate_ref, wff2_ref,
               out_ref):
    pid = pl.program_id(0)
    g = g_ref[...]
    tab = tab_ref[...]

    sdst = jnp.dot(tab, wadst_ref[...], preferred_element_type=jnp.float32)
    ssrc = jnp.dot(g, wasrc_ref[...], preferred_element_type=jnp.float32)

    vec = g[:, 0:3].reshape(AT_R, K, 3) - tab[:, 0:3].reshape(AT_R, 1, 3)
    dist2 = jnp.sum(vec * vec, axis=2, keepdims=True)
    dist = jnp.sqrt(dist2)
    sel = dist > 1e-3

    distf = dist.reshape(AT_E, 1)
    mu = lax.broadcasted_iota(jnp.int32, (1, 16), 1).astype(jnp.float32) \
        * (20.0 / 15.0)
    rbf = jnp.exp(-(((distf - mu) / _SIGMA) ** 2))
    dstf = ((lax.broadcasted_iota(jnp.int32, (AT_E, 1), 0) // K)
            + pid * AT_R).astype(jnp.float32)
    dd = sf_ref[...] - dstf
    freq = jnp.exp(lax.broadcasted_iota(jnp.int32, (1, 8), 1).astype(jnp.float32)
                   * jnp.float32(-2.0 / 16.0 * np.log(10000.0)))
    ang = dd * freq
    ef = jnp.concatenate([rbf, jnp.cos(ang), jnp.sin(ang)], axis=1)
    eh_lin = jnp.dot(ef, wedge_ref[...],
                     preferred_element_type=jnp.float32) + bedge_ref[...]
    eh = eh_lin * jax.nn.sigmoid(eh_lin)
    seh = jnp.dot(eh, waeh_ref[...], preferred_element_type=jnp.float32)

    sc_lin = (ssrc + seh).reshape(AT_R, K, NHEADS) + sdst.reshape(AT_R, 1, NHEADS)
    scores = jnp.where(sc_lin >= 0, sc_lin, 0.2 * sc_lin)
    scores = jnp.where(sel, scores, -jnp.inf)
    m = jnp.max(scores, axis=1, keepdims=True)
    e = jnp.where(sel, jnp.exp(scores - m), 0.0)
    den = jnp.sum(e, axis=1, keepdims=True)
    alpha = e / (den + 1e-9)

    amat = jnp.dot(alpha.reshape(AT_E, NHEADS), phead_ref[...],
                   preferred_element_type=jnp.float32)
    v = jnp.dot(g, wval_ref[...], preferred_element_type=jnp.float32)
    agg = jnp.sum((v * amat).reshape(AT_R, K, NCOEF * NHEADS * VCH), axis=1)
    attn = jnp.dot(agg, wproj_ref[...], preferred_element_type=jnp.float32)

    out0 = ne_ref[...] + attn
    h1 = jnp.dot(out0, wff1_ref[...], preferred_element_type=jnp.float32)
    sig = jax.nn.sigmoid(h1[:, 0:HC])
    tile = jnp.dot(sig, tgate_ref[...], preferred_element_type=jnp.float32)
    hg = h1 * tile
    out_ref[...] = out0 + jnp.dot(hg, wff2_ref[...],
                                  preferred_element_type=jnp.float32)


def _attn(gathered, table, ne_flat, src_f, weights):
    full = lambda shape: pl.BlockSpec(shape, lambda i: (0,) * len(shape))
    return pl.pallas_call(
        _attn_body,
        grid=(NP // AT_R,),
        in_specs=[
            pl.BlockSpec((AT_E, D), lambda i: (i, 0)),
            pl.BlockSpec((AT_R, D), lambda i: (i, 0)),
            pl.BlockSpec((AT_R, NCOEF * HC), lambda i: (i, 0)),
            pl.BlockSpec((AT_E, 1), lambda i: (i, 0)),
        ] + [full(w.shape) for w in weights],
        out_specs=pl.BlockSpec((AT_R, NCOEF * HC), lambda i: (i, 0)),
        out_shape=jax.ShapeDtypeStruct((NP, NCOEF * HC), jnp.float32),
    )(gathered, table, ne_flat, src_f, *weights)


def _build_weights(W_edge, b_edge, W_alpha, W_value, W_proj, W_ff1, W_ff2):
    f32 = jnp.float32
    wadst = jnp.zeros((D, NHEADS), f32).at[3:3 + CTOT].set(W_alpha[0:CTOT])
    wasrc = jnp.zeros((D, NHEADS), f32).at[3:3 + CTOT].set(W_alpha[CTOT:2 * CTOT])
    waeh = W_alpha[2 * CTOT:]
    V = NHEADS * VCH
    wval = jnp.zeros((D, NCOEF * V), f32)
    wproj = jnp.zeros((NCOEF * V, NCOEF * HC), f32)
    wff1 = jnp.zeros((NCOEF * HC, NCOEF * HC), f32)
    wff2 = jnp.zeros((NCOEF * HC, NCOEF * HC), f32)
    for l in range(NCOEF):
        wval = wval.at[3 + l * CTOT:3 + (l + 1) * CTOT,
                       l * V:(l + 1) * V].set(W_value)
        wproj = wproj.at[l * V:(l + 1) * V, l * HC:(l + 1) * HC].set(W_proj)
        wff1 = wff1.at[l * HC:(l + 1) * HC, l * HC:(l + 1) * HC].set(W_ff1)
        wff2 = wff2.at[l * HC:(l + 1) * HC, l * HC:(l + 1) * HC].set(W_ff2)
    ph = np.zeros((NHEADS, NCOEF * V), np.float32)
    for mcol in range(NCOEF * V):
        ph[(mcol % V) // VCH, mcol] = 1.0
    tg = np.zeros((HC, NCOEF * HC), np.float32)
    for l in range(NCOEF):
        for c in range(HC):
            tg[c, l * HC + c] = 1.0
    return [wadst, wasrc, W_edge, b_edge.reshape(1, HC), waeh,
            wval, jnp.asarray(ph), wproj, wff1, jnp.asarray(tg), wff2]


def kernel(trans, rots, node_emb, batch, x_mask, noising_mask,
           W_edge, b_edge, W_alpha, W_value, W_proj, W_ff1, W_ff2):
    f32 = jnp.float32
    frame_atoms = jnp.einsum('nij,aj->nai', rots, jnp.asarray(_IDEAL)) \
        + trans[:, None, :]
    fused = jnp.zeros((N, NCOEF, CTOT), f32)
    fused = fused.at[..., :HC].set(node_emb)
    fused = fused.at[:, 1:4, HC:].set(jnp.swapaxes(frame_atoms, -1, -2))
    editable = noising_mask & (~x_mask)
    fused = fused.at[:, 0, CTOT - 1].set(editable.astype(f32))

    table = jnp.concatenate(
        [trans, fused.reshape(N, NCOEF * CTOT),
         jnp.zeros((N, D - 3 - NCOEF * CTOT), f32)], axis=1)
    table = jnp.concatenate([table, jnp.zeros((NP - N, D), f32)], axis=0)

    trans_pad = table[:, 0:3]
    transT3 = trans_pad.T.reshape(3, KNN_NCH, KNN_CCH).transpose(1, 0, 2)
    batch_pad = jnp.concatenate(
        [batch.astype(jnp.int32), jnp.full((NP - N,), -1, jnp.int32)])
    batch2d = batch_pad.reshape(NP, 1)
    batchT3 = batch_pad.reshape(KNN_NCH, 1, KNN_CCH)

    nbr = _knn(trans_pad, transT3, batch2d, batchT3)
    idx_flat = nbr.reshape(NE)
    gathered = _sc_gather(table, idx_flat)

    src_f = nbr.reshape(NE, 1).astype(f32)
    ne_flat = jnp.concatenate(
        [node_emb.reshape(N, NCOEF * HC), jnp.zeros((NP - N, NCOEF * HC), f32)],
        axis=0)
    weights = _build_weights(W_edge, b_edge, W_alpha, W_value, W_proj,
                             W_ff1, W_ff2)
    out = _attn(gathered, table, ne_flat, src_f, weights)
    return out[:N].reshape(N, NCOEF, HC)


# speedup vs baseline: 1.2549x; 1.2549x over previous
import functools

import jax
import jax.numpy as jnp
import numpy as np
from jax import lax
from jax.experimental import pallas as pl
from jax.experimental.pallas import tpu as pltpu
from jax.experimental.pallas import tpu_sc as plsc

N = 10000
K = 30
NP = 10240
NE = NP * K
NCOEF = 9
HC = 32
CTOT = 35
NHEADS = 8
VCH = 8
D = 384

KNN_R = 256
KNN_CCH = 512
KNN_NCH = NP // KNN_CCH

AT_R = 64
AT_E = AT_R * K

SC_NC = 2
SC_NS = 16
SC_NW = SC_NC * SC_NS
SC_PER_W = NE // SC_NW
SC_CH = 320
SC_NIT = SC_PER_W // SC_CH

_SIGMA = np.float32(20.0 / 16)
_IDEAL = np.array([[-0.525, 1.363, 0.0], [0.0, 0.0, 0.0], [1.526, 0.0, 0.0]],
                  dtype=np.float32)


def _knn_body(tb_ref, tT_ref, bb_ref, bT_ref, out_ref, d2_ref, idxs_ref,
              cmin_ref, idx_scr):
    pid = pl.program_id(0)
    xb = tb_ref[...]
    bb = bb_ref[...]
    rowid = lax.broadcasted_iota(jnp.int32, (KNN_R, 1), 0) + pid * KNN_R
    liota = lax.broadcasted_iota(jnp.int32, (KNN_R, KNN_CCH), 1)

    def fill(c, _):
        x0 = xb[:, 0:1] - tT_ref[c, 0:1, :]
        x1 = xb[:, 1:2] - tT_ref[c, 1:2, :]
        x2 = xb[:, 2:3] - tT_ref[c, 2:3, :]
        d2 = (x0 * x0 + x1 * x1) + x2 * x2
        bm = bb != bT_ref[c, 0:1, :]
        d2 = jnp.where(bm, 1e9, d2)
        d2 = jnp.where(liota == rowid - c * KNN_CCH, 1e9, d2)
        d2_ref[c] = d2
        cmin_ref[c] = jnp.min(d2, axis=1, keepdims=True)
        return 0

    lax.fori_loop(0, KNN_NCH, fill, 0)

    ciota = lax.broadcasted_iota(jnp.int32, (KNN_NCH, KNN_R, 1), 0)

    def extract_one(i, _):
        cmin3 = cmin_ref[...]
        m = jnp.min(cmin3, axis=0)
        cbest = jnp.min(jnp.where(cmin3 == m[None], ciota, 1 << 30), axis=0)

        def scan(c, _):
            @pl.when(jnp.any(cbest == c))
            def _():
                ch = d2_ref[c]
                rowsel = cbest == c
                lidx = jnp.min(
                    jnp.where((ch == m) & rowsel, liota, 1 << 30),
                    axis=1, keepdims=True)
                ch_new = jnp.where(liota == lidx, 2e9, ch)
                d2_ref[c] = ch_new
                cmin_ref[c] = jnp.min(ch_new, axis=1, keepdims=True)
                idx_scr[...] = jnp.where(rowsel, lidx + c * KNN_CCH,
                                         idx_scr[...])
            return 0

        lax.fori_loop(0, KNN_NCH, scan, 0)
        idxs_ref[i] = idx_scr[...]
        return 0

    lax.fori_loop(0, K, extract_one, 0)
    for k in range(K):
        out_ref[:, k:k + 1] = idxs_ref[k]


def _knn(trans_pad, transT3, batch2d, batchT3):
    return pl.pallas_call(
        _knn_body,
        grid=(NP // KNN_R,),
        in_specs=[
            pl.BlockSpec((KNN_R, 3), lambda i: (i, 0)),
            pl.BlockSpec((KNN_NCH, 3, KNN_CCH), lambda i: (0, 0, 0)),
            pl.BlockSpec((KNN_R, 1), lambda i: (i, 0)),
            pl.BlockSpec((KNN_NCH, 1, KNN_CCH), lambda i: (0, 0, 0)),
        ],
        out_specs=pl.BlockSpec((KNN_R, K), lambda i: (i, 0)),
        out_shape=jax.ShapeDtypeStruct((NP, K), jnp.int32),
        scratch_shapes=[pltpu.VMEM((KNN_NCH, KNN_R, KNN_CCH), jnp.float32),
                        pltpu.VMEM((K, KNN_R, 1), jnp.int32),
                        pltpu.VMEM((KNN_NCH, KNN_R, 1), jnp.float32),
                        pltpu.VMEM((KNN_R, 1), jnp.int32)],
    )(trans_pad, transT3, batch2d, batchT3)


def _sc_gather_body(table_hbm, idx_hbm, out_hbm, idx_v, rows_v, sem):
    wid = lax.axis_index("s") * SC_NC + lax.axis_index("c")
    base = wid * SC_PER_W

    def body(it, _):
        off = base + it * SC_CH
        pltpu.sync_copy(idx_hbm.at[pl.ds(off, SC_CH)], idx_v)
        pltpu.async_copy(table_hbm.at[idx_v], rows_v, sem).wait()
        pltpu.sync_copy(rows_v, out_hbm.at[pl.ds(off, SC_CH)])
        return 0

    lax.fori_loop(0, SC_NIT, body, 0)


def _sc_gather(table, idx_flat):
    mesh = plsc.VectorSubcoreMesh(core_axis_name="c", subcore_axis_name="s")
    f = functools.partial(
        pl.kernel,
        mesh=mesh,
        out_type=jax.ShapeDtypeStruct((NE, D), jnp.float32),
        scratch_types=[
            pltpu.VMEM((SC_CH,), jnp.int32),
            pltpu.VMEM((SC_CH, D), jnp.float32),
            pltpu.SemaphoreType.DMA,
        ],
    )(_sc_gather_body)
    return f(table, idx_flat)


def _attn_body(g_ref, tab_ref, ne_ref, sf_ref,
               wadst_ref, wasrc_ref, we1_ref, we2_ref, we3_ref, bedge_ref,
               waeh_ref, wval_ref, phead_ref, wproj_ref, wff1_ref, tgate_ref,
               wff2_ref, out_ref):
    pid = pl.program_id(0)
    g = g_ref[...]
    tab = tab_ref[...]

    sdst = jnp.dot(tab, wadst_ref[...], preferred_element_type=jnp.float32)
    ssrc = jnp.dot(g, wasrc_ref[...], preferred_element_type=jnp.float32)

    vec = g[:, 0:3].reshape(AT_R, K, 3) - tab[:, 0:3].reshape(AT_R, 1, 3)
    dist2 = jnp.sum(vec * vec, axis=2, keepdims=True)
    dist = jnp.sqrt(dist2)
    sel = dist > 1e-3

    distf = dist.reshape(AT_E, 1)
    mu = lax.broadcasted_iota(jnp.int32, (1, 16), 1).astype(jnp.float32) \
        * (20.0 / 15.0)
    rbf = jnp.exp(-(((distf - mu) / _SIGMA) ** 2))
    dstf = (lax.broadcasted_iota(jnp.int32, (AT_R, K, 1), 0).reshape(AT_E, 1)
            + pid * AT_R).astype(jnp.float32)
    dd = sf_ref[...] - dstf
    freq = jnp.exp(lax.broadcasted_iota(jnp.int32, (1, 8), 1).astype(jnp.float32)
                   * jnp.float32(-2.0 / 16.0 * np.log(10000.0)))
    ang = dd * freq
    eh_lin = (jnp.dot(rbf, we1_ref[...], preferred_element_type=jnp.float32)
              + jnp.dot(jnp.cos(ang), we2_ref[...],
                        preferred_element_type=jnp.float32)
              + jnp.dot(jnp.sin(ang), we3_ref[...],
                        preferred_element_type=jnp.float32)
              + bedge_ref[...])
    eh = eh_lin * jax.nn.sigmoid(eh_lin)
    seh = jnp.dot(eh, waeh_ref[...], preferred_element_type=jnp.float32)

    sc_lin = (ssrc + seh).reshape(AT_R, K, NHEADS) + sdst.reshape(AT_R, 1, NHEADS)
    scores = jnp.where(sc_lin >= 0, sc_lin, 0.2 * sc_lin)
    scores = jnp.where(sel, scores, -jnp.inf)
    m = jnp.max(scores, axis=1, keepdims=True)
    e = jnp.where(sel, jnp.exp(scores - m), 0.0)
    den = jnp.sum(e, axis=1, keepdims=True)
    alpha = e / (den + 1e-9)

    amat = jnp.dot(alpha.reshape(AT_E, NHEADS), phead_ref[...],
                   preferred_element_type=jnp.float32)
    v = jnp.dot(g, wval_ref[...], preferred_element_type=jnp.float32)
    agg = jnp.sum((v * amat).reshape(AT_R, K, NCOEF * NHEADS * VCH), axis=1)
    attn = jnp.dot(agg, wproj_ref[...], preferred_element_type=jnp.float32)

    out0 = ne_ref[...] + attn
    h1 = jnp.dot(out0, wff1_ref[...], preferred_element_type=jnp.float32)
    sig = jax.nn.sigmoid(h1[:, 0:HC])
    tile = jnp.dot(sig, tgate_ref[...], preferred_element_type=jnp.float32)
    hg = h1 * tile
    out_ref[...] = out0 + jnp.dot(hg, wff2_ref[...],
                                  preferred_element_type=jnp.float32)


def _attn(gathered, table, ne_flat, src_f, weights):
    full = lambda shape: pl.BlockSpec(shape, lambda i: (0,) * len(shape))
    return pl.pallas_call(
        _attn_body,
        grid=(NP // AT_R,),
        in_specs=[
            pl.BlockSpec((AT_E, D), lambda i: (i, 0)),
            pl.BlockSpec((AT_R, D), lambda i: (i, 0)),
            pl.BlockSpec((AT_R, NCOEF * HC), lambda i: (i, 0)),
            pl.BlockSpec((AT_E, 1), lambda i: (i, 0)),
        ] + [full(w.shape) for w in weights],
        out_specs=pl.BlockSpec((AT_R, NCOEF * HC), lambda i: (i, 0)),
        out_shape=jax.ShapeDtypeStruct((NP, NCOEF * HC), jnp.float32),
    )(gathered, table, ne_flat, src_f, *weights)


def _build_weights(W_edge, b_edge, W_alpha, W_value, W_proj, W_ff1, W_ff2):
    f32 = jnp.float32
    wadst = jnp.zeros((D, NHEADS), f32).at[3:3 + CTOT].set(W_alpha[0:CTOT])
    wasrc = jnp.zeros((D, NHEADS), f32).at[3:3 + CTOT].set(W_alpha[CTOT:2 * CTOT])
    waeh = W_alpha[2 * CTOT:]
    V = NHEADS * VCH
    wval = jnp.zeros((D, NCOEF * V), f32)
    wproj = jnp.zeros((NCOEF * V, NCOEF * HC), f32)
    wff1 = jnp.zeros((NCOEF * HC, NCOEF * HC), f32)
    wff2 = jnp.zeros((NCOEF * HC, NCOEF * HC), f32)
    for l in range(NCOEF):
        wval = wval.at[3 + l * CTOT:3 + (l + 1) * CTOT,
                       l * V:(l + 1) * V].set(W_value)
        wproj = wproj.at[l * V:(l + 1) * V, l * HC:(l + 1) * HC].set(W_proj)
        wff1 = wff1.at[l * HC:(l + 1) * HC, l * HC:(l + 1) * HC].set(W_ff1)
        wff2 = wff2.at[l * HC:(l + 1) * HC, l * HC:(l + 1) * HC].set(W_ff2)
    ph = np.zeros((NHEADS, NCOEF * V), np.float32)
    for mcol in range(NCOEF * V):
        ph[(mcol % V) // VCH, mcol] = 1.0
    tg = np.zeros((HC, NCOEF * HC), np.float32)
    for l in range(NCOEF):
        for c in range(HC):
            tg[c, l * HC + c] = 1.0
    return [wadst, wasrc, W_edge[0:16], W_edge[16:24], W_edge[24:32],
            b_edge.reshape(1, HC), waeh,
            wval, jnp.asarray(ph), wproj, wff1, jnp.asarray(tg), wff2]


def kernel(trans, rots, node_emb, batch, x_mask, noising_mask,
           W_edge, b_edge, W_alpha, W_value, W_proj, W_ff1, W_ff2):
    f32 = jnp.float32
    frame_atoms = jnp.einsum('nij,aj->nai', rots, jnp.asarray(_IDEAL)) \
        + trans[:, None, :]
    fused = jnp.zeros((N, NCOEF, CTOT), f32)
    fused = fused.at[..., :HC].set(node_emb)
    fused = fused.at[:, 1:4, HC:].set(jnp.swapaxes(frame_atoms, -1, -2))
    editable = noising_mask & (~x_mask)
    fused = fused.at[:, 0, CTOT - 1].set(editable.astype(f32))

    table = jnp.concatenate(
        [trans, fused.reshape(N, NCOEF * CTOT),
         jnp.zeros((N, D - 3 - NCOEF * CTOT), f32)], axis=1)
    table = jnp.concatenate([table, jnp.zeros((NP - N, D), f32)], axis=0)

    trans_pad = table[:, 0:3]
    transT3 = trans_pad.T.reshape(3, KNN_NCH, KNN_CCH).transpose(1, 0, 2)
    batch_pad = jnp.concatenate(
        [batch.astype(jnp.int32), jnp.full((NP - N,), -1, jnp.int32)])
    batch2d = batch_pad.reshape(NP, 1)
    batchT3 = batch_pad.reshape(KNN_NCH, 1, KNN_CCH)

    nbr = _knn(trans_pad, transT3, batch2d, batchT3)
    idx_flat = nbr.reshape(NE)
    gathered = _sc_gather(table, idx_flat)

    src_f = nbr.reshape(NE, 1).astype(f32)
    ne_flat = jnp.concatenate(
        [node_emb.reshape(N, NCOEF * HC), jnp.zeros((NP - N, NCOEF * HC), f32)],
        axis=0)
    weights = _build_weights(W_edge, b_edge, W_alpha, W_value, W_proj,
                             W_ff1, W_ff2)
    out = _attn(gathered, table, ne_flat, src_f, weights)
    return out[:N].reshape(N, NCOEF, HC)


# speedup vs baseline: 1.7534x; 1.3972x over previous
import functools

import jax
import jax.numpy as jnp
import numpy as np
from jax import lax
from jax.experimental import pallas as pl
from jax.experimental.pallas import tpu as pltpu
from jax.experimental.pallas import tpu_sc as plsc

N = 10000
K = 30
NP = 10240
NE = NP * K
NCOEF = 9
HC = 32
CTOT = 35
NHEADS = 8
VCH = 8
D = 384

KNN_R = 256
KNN_CCH = 1024
KNN_NCH = NP // KNN_CCH

AT_R = 64
AT_E = AT_R * K

SC_NC = 2
SC_NS = 16
SC_NW = SC_NC * SC_NS
SC_PER_W = NE // SC_NW
SC_CH = 320
SC_NIT = SC_PER_W // SC_CH

_SIGMA = np.float32(20.0 / 16)
_IDEAL = np.array([[-0.525, 1.363, 0.0], [0.0, 0.0, 0.0], [1.526, 0.0, 0.0]],
                  dtype=np.float32)


def _knn_body(tb_ref, tT_ref, bb_ref, bT_ref, out_ref, d2_ref, idxs_ref,
              cmin_ref, idx_scr):
    pid = pl.program_id(0)
    xb = tb_ref[...]
    bb = bb_ref[...]
    rowid = lax.broadcasted_iota(jnp.int32, (KNN_R, 1), 0) + pid * KNN_R
    liota = lax.broadcasted_iota(jnp.int32, (KNN_R, KNN_CCH), 1)

    def fill(c, _):
        x0 = xb[:, 0:1] - tT_ref[c, 0:1, :]
        x1 = xb[:, 1:2] - tT_ref[c, 1:2, :]
        x2 = xb[:, 2:3] - tT_ref[c, 2:3, :]
        d2 = (x0 * x0 + x1 * x1) + x2 * x2
        bm = bb != bT_ref[c, 0:1, :]
        d2 = jnp.where(bm, 1e9, d2)
        d2 = jnp.where(liota == rowid - c * KNN_CCH, 1e9, d2)
        d2_ref[c] = d2
        cmin_ref[c] = jnp.min(d2, axis=1, keepdims=True)
        return 0

    lax.fori_loop(0, KNN_NCH, fill, 0)

    ciota = lax.broadcasted_iota(jnp.int32, (KNN_NCH, KNN_R, 1), 0)

    def extract_one(i, _):
        cmin3 = cmin_ref[...]
        m = jnp.min(cmin3, axis=0)
        cbest = jnp.min(jnp.where(cmin3 == m[None], ciota, 1 << 30), axis=0)

        def scan(c, _):
            @pl.when(jnp.any(cbest == c))
            def _():
                ch = d2_ref[c]
                rowsel = cbest == c
                lidx = jnp.min(
                    jnp.where((ch == m) & rowsel, liota, 1 << 30),
                    axis=1, keepdims=True)
                ch_new = jnp.where(liota == lidx, 2e9, ch)
                d2_ref[c] = ch_new
                cmin_ref[c] = jnp.min(ch_new, axis=1, keepdims=True)
                idx_scr[...] = jnp.where(rowsel, lidx + c * KNN_CCH,
                                         idx_scr[...])
            return 0

        lax.fori_loop(0, KNN_NCH, scan, 0)
        idxs_ref[i] = idx_scr[...]
        return 0

    lax.fori_loop(0, K, extract_one, 0)
    for k in range(K):
        out_ref[:, k:k + 1] = idxs_ref[k]


def _knn(trans_pad, transT3, batch2d, batchT3):
    return pl.pallas_call(
        _knn_body,
        grid=(NP // KNN_R,),
        in_specs=[
            pl.BlockSpec((KNN_R, 3), lambda i: (i, 0)),
            pl.BlockSpec((KNN_NCH, 3, KNN_CCH), lambda i: (0, 0, 0)),
            pl.BlockSpec((KNN_R, 1), lambda i: (i, 0)),
            pl.BlockSpec((KNN_NCH, 1, KNN_CCH), lambda i: (0, 0, 0)),
        ],
        out_specs=pl.BlockSpec((KNN_R, K), lambda i: (i, 0)),
        out_shape=jax.ShapeDtypeStruct((NP, K), jnp.int32),
        scratch_shapes=[pltpu.VMEM((KNN_NCH, KNN_R, KNN_CCH), jnp.float32),
                        pltpu.VMEM((K, KNN_R, 1), jnp.int32),
                        pltpu.VMEM((KNN_NCH, KNN_R, 1), jnp.float32),
                        pltpu.VMEM((KNN_R, 1), jnp.int32)],
    )(trans_pad, transT3, batch2d, batchT3)


def _sc_gather_body(table_hbm, idx_hbm, out_hbm, idx_v, rows_v, sem):
    wid = lax.axis_index("s") * SC_NC + lax.axis_index("c")
    base = wid * SC_PER_W

    def body(it, _):
        off = base + it * SC_CH
        pltpu.sync_copy(idx_hbm.at[pl.ds(off, SC_CH)], idx_v)
        pltpu.async_copy(table_hbm.at[idx_v], rows_v, sem).wait()
        pltpu.sync_copy(rows_v, out_hbm.at[pl.ds(off, SC_CH)])
        return 0

    lax.fori_loop(0, SC_NIT, body, 0)


def _sc_gather(table, idx_flat):
    mesh = plsc.VectorSubcoreMesh(core_axis_name="c", subcore_axis_name="s")
    f = functools.partial(
        pl.kernel,
        mesh=mesh,
        out_type=jax.ShapeDtypeStruct((NE, D), jnp.float32),
        scratch_types=[
            pltpu.VMEM((SC_CH,), jnp.int32),
            pltpu.VMEM((SC_CH, D), jnp.float32),
            pltpu.SemaphoreType.DMA,
        ],
    )(_sc_gather_body)
    return f(table, idx_flat)


def _attn_body(g_ref, tab_ref, ne_ref, sf_ref,
               wadst_ref, wasrc_ref, we1_ref, we2_ref, we3_ref, bedge_ref,
               waeh_ref, wval_ref, phead_ref, wproj_ref, wff1_ref, tgate_ref,
               wff2_ref, out_ref):
    pid = pl.program_id(0)
    g = g_ref[...]
    tab = tab_ref[...]

    sdst = jnp.dot(tab, wadst_ref[...], preferred_element_type=jnp.float32)
    ssrc = jnp.dot(g, wasrc_ref[...], preferred_element_type=jnp.float32)

    vec = g[:, 0:3].reshape(AT_R, K, 3) - tab[:, 0:3].reshape(AT_R, 1, 3)
    dist2 = jnp.sum(vec * vec, axis=2, keepdims=True)
    dist = jnp.sqrt(dist2)
    sel = dist > 1e-3

    distf = dist.reshape(AT_E, 1)
    mu = lax.broadcasted_iota(jnp.int32, (1, 16), 1).astype(jnp.float32) \
        * (20.0 / 15.0)
    rbf = jnp.exp(-(((distf - mu) / _SIGMA) ** 2))
    dstf = (lax.broadcasted_iota(jnp.int32, (AT_R, K, 1), 0).reshape(AT_E, 1)
            + pid * AT_R).astype(jnp.float32)
    dd = sf_ref[...] - dstf
    freq = jnp.exp(lax.broadcasted_iota(jnp.int32, (1, 8), 1).astype(jnp.float32)
                   * jnp.float32(-2.0 / 16.0 * np.log(10000.0)))
    ang = dd * freq
    eh_lin = (jnp.dot(rbf, we1_ref[...], preferred_element_type=jnp.float32)
              + jnp.dot(jnp.cos(ang), we2_ref[...],
                        preferred_element_type=jnp.float32)
              + jnp.dot(jnp.sin(ang), we3_ref[...],
                        preferred_element_type=jnp.float32)
              + bedge_ref[...])
    eh = eh_lin * jax.nn.sigmoid(eh_lin)
    seh = jnp.dot(eh, waeh_ref[...], preferred_element_type=jnp.float32)

    sc_lin = (ssrc + seh).reshape(AT_R, K, NHEADS) + sdst.reshape(AT_R, 1, NHEADS)
    scores = jnp.where(sc_lin >= 0, sc_lin, 0.2 * sc_lin)
    scores = jnp.where(sel, scores, -jnp.inf)
    m = jnp.max(scores, axis=1, keepdims=True)
    e = jnp.where(sel, jnp.exp(scores - m), 0.0)
    den = jnp.sum(e, axis=1, keepdims=True)
    alpha = e / (den + 1e-9)

    amat = jnp.dot(alpha.reshape(AT_E, NHEADS), phead_ref[...],
                   preferred_element_type=jnp.float32)
    v = jnp.dot(g, wval_ref[...], preferred_element_type=jnp.float32)
    agg = jnp.sum((v * amat).reshape(AT_R, K, NCOEF * NHEADS * VCH), axis=1)
    attn = jnp.dot(agg, wproj_ref[...], preferred_element_type=jnp.float32)

    out0 = ne_ref[...] + attn
    h1 = jnp.dot(out0, wff1_ref[...], preferred_element_type=jnp.float32)
    sig = jax.nn.sigmoid(h1[:, 0:HC])
    tile = jnp.dot(sig, tgate_ref[...], preferred_element_type=jnp.float32)
    hg = h1 * tile
    out_ref[...] = out0 + jnp.dot(hg, wff2_ref[...],
                                  preferred_element_type=jnp.float32)


def _attn(gathered, table, ne_flat, src_f, weights):
    full = lambda shape: pl.BlockSpec(shape, lambda i: (0,) * len(shape))
    return pl.pallas_call(
        _attn_body,
        grid=(NP // AT_R,),
        in_specs=[
            pl.BlockSpec((AT_E, D), lambda i: (i, 0)),
            pl.BlockSpec((AT_R, D), lambda i: (i, 0)),
            pl.BlockSpec((AT_R, NCOEF * HC), lambda i: (i, 0)),
            pl.BlockSpec((AT_E, 1), lambda i: (i, 0)),
        ] + [full(w.shape) for w in weights],
        out_specs=pl.BlockSpec((AT_R, NCOEF * HC), lambda i: (i, 0)),
        out_shape=jax.ShapeDtypeStruct((NP, NCOEF * HC), jnp.float32),
    )(gathered, table, ne_flat, src_f, *weights)


def _build_weights(W_edge, b_edge, W_alpha, W_value, W_proj, W_ff1, W_ff2):
    f32 = jnp.float32
    wadst = jnp.zeros((D, NHEADS), f32).at[3:3 + CTOT].set(W_alpha[0:CTOT])
    wasrc = jnp.zeros((D, NHEADS), f32).at[3:3 + CTOT].set(W_alpha[CTOT:2 * CTOT])
    waeh = W_alpha[2 * CTOT:]
    V = NHEADS * VCH
    wval = jnp.zeros((D, NCOEF * V), f32)
    wproj = jnp.zeros((NCOEF * V, NCOEF * HC), f32)
    wff1 = jnp.zeros((NCOEF * HC, NCOEF * HC), f32)
    wff2 = jnp.zeros((NCOEF * HC, NCOEF * HC), f32)
    for l in range(NCOEF):
        wval = wval.at[3 + l * CTOT:3 + (l + 1) * CTOT,
                       l * V:(l + 1) * V].set(W_value)
        wproj = wproj.at[l * V:(l + 1) * V, l * HC:(l + 1) * HC].set(W_proj)
        wff1 = wff1.at[l * HC:(l + 1) * HC, l * HC:(l + 1) * HC].set(W_ff1)
        wff2 = wff2.at[l * HC:(l + 1) * HC, l * HC:(l + 1) * HC].set(W_ff2)
    ph = np.zeros((NHEADS, NCOEF * V), np.float32)
    for mcol in range(NCOEF * V):
        ph[(mcol % V) // VCH, mcol] = 1.0
    tg = np.zeros((HC, NCOEF * HC), np.float32)
    for l in range(NCOEF):
        for c in range(HC):
            tg[c, l * HC + c] = 1.0
    return [wadst, wasrc, W_edge[0:16], W_edge[16:24], W_edge[24:32],
            b_edge.reshape(1, HC), waeh,
            wval, jnp.asarray(ph), wproj, wff1, jnp.asarray(tg), wff2]


def kernel(trans, rots, node_emb, batch, x_mask, noising_mask,
           W_edge, b_edge, W_alpha, W_value, W_proj, W_ff1, W_ff2):
    f32 = jnp.float32
    frame_atoms = jnp.einsum('nij,aj->nai', rots, jnp.asarray(_IDEAL)) \
        + trans[:, None, :]
    fused = jnp.zeros((N, NCOEF, CTOT), f32)
    fused = fused.at[..., :HC].set(node_emb)
    fused = fused.at[:, 1:4, HC:].set(jnp.swapaxes(frame_atoms, -1, -2))
    editable = noising_mask & (~x_mask)
    fused = fused.at[:, 0, CTOT - 1].set(editable.astype(f32))

    table = jnp.concatenate(
        [trans, fused.reshape(N, NCOEF * CTOT),
         jnp.zeros((N, D - 3 - NCOEF * CTOT), f32)], axis=1)
    table = jnp.concatenate([table, jnp.zeros((NP - N, D), f32)], axis=0)

    trans_pad = table[:, 0:3]
    transT3 = trans_pad.T.reshape(3, KNN_NCH, KNN_CCH).transpose(1, 0, 2)
    batch_pad = jnp.concatenate(
        [batch.astype(jnp.int32), jnp.full((NP - N,), -1, jnp.int32)])
    batch2d = batch_pad.reshape(NP, 1)
    batchT3 = batch_pad.reshape(KNN_NCH, 1, KNN_CCH)

    nbr = _knn(trans_pad, transT3, batch2d, batchT3)
    idx_flat = nbr.reshape(NE)
    gathered = _sc_gather(table, idx_flat)

    src_f = nbr.reshape(NE, 1).astype(f32)
    ne_flat = jnp.concatenate(
        [node_emb.reshape(N, NCOEF * HC), jnp.zeros((NP - N, NCOEF * HC), f32)],
        axis=0)
    weights = _build_weights(W_edge, b_edge, W_alpha, W_value, W_proj,
                             W_ff1, W_ff2)
    out = _attn(gathered, table, ne_flat, src_f, weights)
    return out[:N].reshape(N, NCOEF, HC)
